# Initial kernel scaffold; baseline (speedup 1.0000x reference)
import jax
import jax.numpy as jnp
from jax.experimental import pallas as pl


def kernel(x, edge_index, batch, Wl1, bl1, Wr1, Wl2, bl2, Wr2, Wc, bc):
    raise NotImplementedError("write your pallas kernel here")


# speedup vs baseline: 2.7515x; 2.7515x over previous
import jax
import jax.numpy as jnp
from jax import lax
from jax.experimental import pallas as pl
from jax.experimental.pallas import tpu as pltpu
from jax.experimental.pallas import tpu_sc as plsc

NSC = 2
NT = 16
EBLK = 128
ZR = 128


def _sc_aggregate(n_pad, n_eblocks, num_tables, with_deg):
    nacc = n_pad + 16
    bpt = n_eblocks // NT
    rpt = n_pad // NT
    assert n_eblocks % NT == 0 and bpt % 8 == 0
    assert n_pad % NT == 0 and rpt % ZR == 0
    passes = num_tables // NSC
    assert num_tables % NSC == 0

    mesh = plsc.VectorSubcoreMesh(core_axis_name="c", subcore_axis_name="s",
                                  num_cores=NSC, num_subcores=NT)

    def body(*refs):
        tables = refs[:num_tables]
        srcb_h, dstb_h, z128, ones128 = refs[num_tables:num_tables + 4]
        outs = refs[num_tables + 4:2 * num_tables + 4]
        rest = refs[2 * num_tables + 4:]
        if with_deg:
            deg0, deg1 = rest[0], rest[1]
            rest = rest[2:]
        (acc, srcb, dstb, rows, sem) = rest

        cid = lax.axis_index("c")
        sid = lax.axis_index("s")
        sb = 8

        def zero_acc():
            for k in range(rpt // ZR):
                zs = pl.ds(pl.multiple_of(sid * rpt + k * ZR, 8), ZR)
                pltpu.sync_copy(z128, acc.at[zs])

        def edge_loop(tbl, base, nblk):
            def stage(s, carry):
                eoff = pl.multiple_of(base + s * sb, 8)
                if tbl is not None:
                    pltpu.sync_copy(srcb_h.at[pl.ds(eoff, sb)], srcb)
                pltpu.sync_copy(dstb_h.at[pl.ds(eoff, sb)], dstb)

                def step(j, carry2):
                    if tbl is not None:
                        pltpu.async_copy(tbl.at[srcb.at[j]], rows, sem).wait()
                    pltpu.sync_copy(rows, acc.at[dstb.at[j]], add=True)
                    return carry2
                lax.fori_loop(0, sb, step, 0)
                return carry
            lax.fori_loop(0, nblk // sb, stage, 0)

        rs = pl.ds(pl.multiple_of(sid * rpt, 8), rpt)
        for p in range(passes):
            zero_acc()
            plsc.subcore_barrier()
            for c in range(NSC):
                t = p * NSC + c

                @pl.when(cid == c)
                def _(t=t):
                    edge_loop(tables[t], sid * bpt, bpt)
            plsc.subcore_barrier()
            for c in range(NSC):
                t = p * NSC + c

                @pl.when(cid == c)
                def _(t=t):
                    pltpu.sync_copy(acc.at[rs], outs[t].at[rs])

        if with_deg:
            zero_acc()
            pltpu.sync_copy(ones128, rows)
            plsc.subcore_barrier()
            half = n_eblocks // NSC
            edge_loop(None, cid * half + sid * (half // NT), half // NT)
            plsc.subcore_barrier()

            @pl.when(cid == 0)
            def _():
                pltpu.sync_copy(acc.at[rs], deg0.at[rs])

            @pl.when(cid == 1)
            def _():
                pltpu.sync_copy(acc.at[rs], deg1.at[rs])

    out_type = tuple(
        [jax.ShapeDtypeStruct((n_pad, 128), jnp.float32)]
        * (num_tables + (2 if with_deg else 0)))
    return pl.kernel(
        body,
        out_type=out_type,
        mesh=mesh,
        scratch_types=[
            pltpu.VMEM_SHARED((nacc, 128), jnp.float32),
            pltpu.VMEM((8, 128), jnp.int32),
            pltpu.VMEM((8, 128), jnp.int32),
            pltpu.VMEM((EBLK, 128), jnp.float32),
            pltpu.SemaphoreType.DMA,
        ],
    )


def _tc_layer1(n_pad, d_in, h_out, bm):
    nb = n_pad // bm
    nchunk = h_out // 128

    def body(a0, a1, d0, d1, x, wl, wr, b, *outs):
        invd = 1.0 / jnp.clip(d0[:, 0:1] + d1[:, 0:1], 1.0, None)
        aggm = jnp.concatenate([a0[...], a1[...]], axis=1) * invd
        h = lax.dot_general(aggm, wl[...], (((1,), (1,)), ((), ())))
        h = h + lax.dot_general(x[...], wr[...], (((1,), (1,)), ((), ())))
        h = jnp.maximum(h + b[...], 0.0)
        for c in range(nchunk):
            outs[c][...] = h[:, c * 128:(c + 1) * 128]

    return pl.pallas_call(
        body,
        grid=(nb,),
        in_specs=[
            pl.BlockSpec((bm, 128), lambda m: (m, 0)),
            pl.BlockSpec((bm, 128), lambda m: (m, 0)),
            pl.BlockSpec((bm, 128), lambda m: (m, 0)),
            pl.BlockSpec((bm, 128), lambda m: (m, 0)),
            pl.BlockSpec((bm, d_in), lambda m: (m, 0)),
            pl.BlockSpec((h_out, d_in), lambda m: (0, 0)),
            pl.BlockSpec((h_out, d_in), lambda m: (0, 0)),
            pl.BlockSpec((1, h_out), lambda m: (0, 0)),
        ],
        out_specs=[pl.BlockSpec((bm, 128), lambda m: (m, 0))] * nchunk,
        out_shape=[jax.ShapeDtypeStruct((n_pad, 128), jnp.float32)] * nchunk,
    )


def _tc_layer2_head(n_pad, h_dim, n_cls, n_graphs, bm):
    nb = n_pad // bm

    def body(a0, a1, a2, a3, d0, d1, h0, h1, h2, h3, bt, wl, wr, b2, wc, bc2,
             out, pooled, cnt):
        m = pl.program_id(0)
        invd = 1.0 / jnp.clip(d0[:, 0:1] + d1[:, 0:1], 1.0, None)
        aggm = jnp.concatenate(
            [a0[...], a1[...], a2[...], a3[...]], axis=1) * invd
        hh = jnp.concatenate([h0[...], h1[...], h2[...], h3[...]], axis=1)
        z = lax.dot_general(aggm, wl[...], (((1,), (1,)), ((), ())))
        z = z + lax.dot_general(hh, wr[...], (((1,), (1,)), ((), ())))
        z = jnp.maximum(z + b2[...], 0.0)
        onehot = (bt[...] == lax.broadcasted_iota(
            jnp.int32, (bm, n_graphs), 1)).astype(jnp.float32)
        ps = lax.dot_general(onehot, z, (((0,), (0,)), ((), ())))
        cs = lax.dot_general(onehot, jnp.ones((bm, 128), jnp.float32),
                             (((0,), (0,)), ((), ())))

        @pl.when(m == 0)
        def _():
            pooled[...] = ps
            cnt[...] = cs

        @pl.when(m > 0)
        def _():
            pooled[...] += ps
            cnt[...] += cs

        @pl.when(m == nb - 1)
        def _():
            pm = pooled[...] / jnp.clip(cnt[...][:, 0:1], 1.0, None)
            out[...] = lax.dot_general(
                pm, wc[...], (((1,), (1,)), ((), ()))) + bc2[...]

    return pl.pallas_call(
        body,
        grid=(nb,),
        in_specs=(
            [pl.BlockSpec((bm, 128), lambda m: (m, 0))] * 4
            + [pl.BlockSpec((bm, 128), lambda m: (m, 0))] * 2
            + [pl.BlockSpec((bm, 128), lambda m: (m, 0))] * 4
            + [
                pl.BlockSpec((bm, 1), lambda m: (m, 0)),
                pl.BlockSpec((h_dim, h_dim), lambda m: (0, 0)),
                pl.BlockSpec((h_dim, h_dim), lambda m: (0, 0)),
                pl.BlockSpec((1, h_dim), lambda m: (0, 0)),
                pl.BlockSpec((n_cls, h_dim), lambda m: (0, 0)),
                pl.BlockSpec((1, n_cls), lambda m: (0, 0)),
            ]),
        out_specs=pl.BlockSpec((n_graphs, n_cls), lambda m: (0, 0)),
        out_shape=jax.ShapeDtypeStruct((n_graphs, n_cls), jnp.float32),
        scratch_shapes=[
            pltpu.VMEM((n_graphs, h_dim), jnp.float32),
            pltpu.VMEM((n_graphs, 128), jnp.float32),
        ],
    )


def kernel(x, edge_index, batch, Wl1, bl1, Wr1, Wl2, bl2, Wr2, Wc, bc):
    n, d = x.shape
    h = Wl1.shape[0]
    c_cls = Wc.shape[0]
    g = 64
    e = edge_index.shape[1]

    npad = ((n + NT * ZR - 1) // (NT * ZR)) * (NT * ZR)
    epad = ((e + NT * 8 * EBLK - 1) // (NT * 8 * EBLK)) * (NT * 8 * EBLK)

    src = edge_index[0].astype(jnp.int32)
    dst = edge_index[1].astype(jnp.int32)
    src = jnp.concatenate([src, jnp.zeros((epad - e,), jnp.int32)])
    dst = jnp.concatenate([dst, jnp.full((epad - e,), npad, jnp.int32)])
    src2d = src.reshape(epad // EBLK, EBLK)
    dst2d = dst.reshape(epad // EBLK, EBLK)

    z128 = jnp.zeros((ZR, 128), jnp.float32)
    ones128 = jnp.ones((EBLK, 128), jnp.float32)

    xc0 = x[:, 0:128]
    xc1 = x[:, 128:256]
    agg0, agg1, deg0, deg1 = _sc_aggregate(npad, epad // EBLK, 2, True)(
        xc0, xc1, src2d, dst2d, z128, ones128)

    xp = jnp.pad(x, ((0, npad - n), (0, 0)))
    h1c = _tc_layer1(npad, d, h, npad // 10)(
        agg0, agg1, deg0, deg1, xp, Wl1, Wr1, bl1.reshape(1, h))

    a2 = _sc_aggregate(npad, epad // EBLK, 4, False)(
        h1c[0], h1c[1], h1c[2], h1c[3], src2d, dst2d, z128, ones128)

    batch2d = jnp.pad(batch.astype(jnp.int32), (0, npad - n),
                      constant_values=g).reshape(npad, 1)
    out = _tc_layer2_head(npad, h, c_cls, g, npad // 10)(
        a2[0], a2[1], a2[2], a2[3], deg0, deg1, h1c[0], h1c[1], h1c[2],
        h1c[3], batch2d, Wl2, Wr2, bl2.reshape(1, h), Wc,
        bc.reshape(1, c_cls))
    return out
